# Initial kernel scaffold; baseline (speedup 1.0000x reference)
import jax
import jax.numpy as jnp
from jax.experimental import pallas as pl


def kernel(mask, input, output):
    raise NotImplementedError("write your pallas kernel here")


# speedup vs baseline: 134.4447x; 134.4447x over previous
import functools

import jax
import jax.numpy as jnp
from jax import lax
from jax.experimental import pallas as pl
from jax.experimental.pallas import tpu as pltpu
from jax.experimental.pallas import tpu_sc as plsc

_DIM = 33
_GRID = _DIM ** 3
_GRID_PAD = 2247 * 16
_LANES = 16
_CH = 4096
_HALF = 131072
_NCHUNK = _HALF // _CH
_INV_BIN = float((_DIM - 1) / 1.000001)

_CORNER_OFF = [(dr, dg, db, dr * _DIM * _DIM + dg * _DIM + db)
               for dr in (0, 1) for dg in (0, 1) for db in (0, 1)]


def _sc_body(inp_ref, msk_ref, out_ref, part_ref,
             grid_v, rbuf, gbuf, bbuf, mbuf, obuf):
    c = lax.axis_index("c")
    s = lax.axis_index("s")
    b = c * 2 + s // 8
    rem = s % 8
    acc = rem // 2
    h = rem % 2
    o_ch = jnp.maximum(acc - 1, 0)
    is_cnt = jnp.broadcast_to(acc == 0, (_LANES,))
    p0 = h * _HALF

    zer = jnp.zeros((_LANES,), jnp.float32)

    def zbody(j, carry):
        grid_v[pl.ds(j * _LANES, _LANES)] = zer
        return carry

    lax.fori_loop(0, _GRID_PAD // _LANES, zbody, 0)

    def vec_body(i, carry):
        off = i * _LANES
        r = rbuf[pl.ds(off, _LANES)]
        g = gbuf[pl.ds(off, _LANES)]
        bl = bbuf[pl.ds(off, _LANES)]
        m = mbuf[pl.ds(off, _LANES)]
        o = obuf[pl.ds(off, _LANES)]

        xr = r * _INV_BIN
        xg = g * _INV_BIN
        xb = bl * _INV_BIN
        ir = xr.astype(jnp.int32)
        ig = xg.astype(jnp.int32)
        ib = xb.astype(jnp.int32)
        irf = ir.astype(jnp.float32)
        igf = ig.astype(jnp.float32)
        ibf = ib.astype(jnp.float32)
        fr = xr - irf
        fg = xg - igf
        fb = xb - ibf

        f = jnp.where(is_cnt, m, m * o)
        a1 = fr * f
        a0 = f - a1
        g1 = fg
        g0 = 1.0 - fg
        b1 = fb
        b0 = 1.0 - fb
        ag = {(0, 0): a0 * g0, (0, 1): a0 * g1,
              (1, 0): a1 * g0, (1, 1): a1 * g1}
        bw = {0: b0, 1: b1}
        base = ir * (_DIM * _DIM) + ig * _DIM + ib
        for dr, dg, db, offc in _CORNER_OFF:
            w = ag[(dr, dg)] * bw[db]
            idx = base + offc
            plsc.addupdate_scatter(grid_v, [idx], w)
        return carry

    npix = 2 * _HALF

    def chunk_body(k, carry):
        base_p = p0 + k * _CH
        inp_base = b * 3 * npix + base_p
        pltpu.sync_copy(inp_ref.at[pl.ds(inp_base, _CH)], rbuf)
        pltpu.sync_copy(inp_ref.at[pl.ds(inp_base + npix, _CH)], gbuf)
        pltpu.sync_copy(inp_ref.at[pl.ds(inp_base + 2 * npix, _CH)], bbuf)
        pltpu.sync_copy(msk_ref.at[pl.ds(b * npix + base_p, _CH)], mbuf)
        pltpu.sync_copy(out_ref.at[pl.ds((b * 3 + o_ch) * npix + base_p, _CH)],
                        obuf)
        lax.fori_loop(0, _CH // _LANES, vec_body, 0)
        return carry

    lax.fori_loop(0, _NCHUNK, chunk_body, 0)

    pltpu.sync_copy(grid_v, part_ref.at[c * 16 + s])


def _make_sc_scatter():
    mesh = plsc.VectorSubcoreMesh(core_axis_name="c", subcore_axis_name="s")
    return pl.kernel(
        _sc_body,
        out_type=jax.ShapeDtypeStruct((32, _GRID_PAD), jnp.float32),
        mesh=mesh,
        compiler_params=pltpu.CompilerParams(needs_layout_passes=False),
        scratch_types=[
            pltpu.VMEM((_GRID_PAD,), jnp.float32),
            pltpu.VMEM((_CH,), jnp.float32),
            pltpu.VMEM((_CH,), jnp.float32),
            pltpu.VMEM((_CH,), jnp.float32),
            pltpu.VMEM((_CH,), jnp.float32),
            pltpu.VMEM((_CH,), jnp.float32),
        ],
    )


def _epi_body(p_ref, lut_ref, cnt_ref):
    for b in range(4):
        cc = b // 2
        sb = (b % 2) * 8

        def g(acc, h):
            return cc * 16 + sb + acc * 2 + h

        cnt = p_ref[g(0, 0), :] + p_ref[g(0, 1), :]
        pos = cnt > 0
        safe = jnp.where(pos, cnt, 1.0)
        for ch in range(3):
            lut = p_ref[g(ch + 1, 0), :] + p_ref[g(ch + 1, 1), :]
            lut_ref[b * 3 + ch, :] = jnp.where(pos, lut / safe, 0.0)
            cnt_ref[b * 3 + ch, :] = cnt


def _epilogue(parts):
    return pl.pallas_call(
        _epi_body,
        out_shape=(jax.ShapeDtypeStruct((12, _GRID_PAD), jnp.float32),
                   jax.ShapeDtypeStruct((12, _GRID_PAD), jnp.float32)),
    )(parts)


def kernel(mask, input, output):
    B, C, H, W = input.shape
    inp2 = input.reshape(B * C * H * W)
    msk2 = mask.reshape(B * H * W)
    out2 = output.reshape(B * C * H * W)
    parts = _make_sc_scatter()(inp2, msk2, out2)
    lutf, cntf = _epilogue(parts)
    lut = lutf[:, :_GRID].reshape(B, C, _DIM, _DIM, _DIM)
    cnt = cntf[:, :_GRID].reshape(B, C, _DIM, _DIM, _DIM)
    return (lut, cnt, output)
